# baseline (speedup 1.0000x reference)
import numpy as np

import jax
import jax.numpy as jnp
from jax import lax
from jax.experimental import pallas as pl
from jax.experimental.pallas import tpu as pltpu
from jax.experimental.pallas import tpu_sc as plsc

R = 128
N = 32768
L = 16
NC = 2
NS = 16
NW = NC * NS
ROWS_PER_W = R // NW
SEG = N // L
NCHUNK = N // L
NST0 = 3 * NCHUNK // 4
NST1 = NCHUNK - NST0
SEG_SHIFT = 11
NBITS = 8
NBUCKET = 1 << NBITS
NPASS = 32 // NBITS
HIST = NBUCKET * L
INT_MIN = np.int32(-2147483648)


def _body(x_hbm, val_hbm, idx_hbm, key_v, pa, pb, hist, abuf, sums, sem):
    wid = lax.axis_index("s") * NC + lax.axis_index("c")
    lane = lax.iota(jnp.int32, L)
    ones = jnp.ones((L,), jnp.int32)
    dmask = np.int32(NBUCKET - 1)

    def do_row(r, row_carry):
        row = wid * ROWS_PER_W + r
        pltpu.sync_copy(x_hbm.at[row], key_v)

        @plsc.parallel_loop(0, NBUCKET, unroll=4)
        def zero0(i):
            hist[pl.ds(i * L, L)] = jnp.zeros((L,), jnp.int32)

        @plsc.parallel_loop(0, NCHUNK, unroll=4)
        def tx(c):
            b = key_v[pl.ds(c * L, L)]
            b = jnp.where(b == INT_MIN, np.int32(0), b)
            k = jnp.where(b < 0, ~b, b ^ INT_MIN)
            key_v[pl.ds(c * L, L)] = k
            addr = (k & dmask) * L + lax.shift_right_logical(c, 7)
            plsc.addupdate_scatter(hist, [addr], ones)

        for p in range(NPASS):
            shift = np.int32(NBITS * p)
            src = (None, pb, pa, pb)[p]
            dst = (pb, pa, pb, pa)[p]

            def iv_at(c):
                if p == 0:
                    return lane * SEG + c
                return src[pl.ds(c * L, L)]

            def addr_at(c):
                kb = plsc.load_gather(key_v, [iv_at(c)])
                d = lax.shift_right_logical(kb, shift) & dmask
                return d * L + lane

            if p > 0:
                @plsc.parallel_loop(0, NBUCKET, unroll=4)
                def zero(i):
                    hist[pl.ds(i * L, L)] = jnp.zeros((L,), jnp.int32)

                @plsc.parallel_loop(0, NST0, unroll=4)
                def aloop0(c):
                    a = addr_at(c)
                    plsc.addupdate_scatter(hist, [a], ones)
                    abuf[pl.ds(c * L, L)] = a

                @plsc.parallel_loop(NST0, NCHUNK, unroll=4)
                def aloop1(c):
                    plsc.addupdate_scatter(hist, [addr_at(c)], ones)

            @plsc.parallel_loop(0, NBUCKET, unroll=4)
            def s1(i):
                sums[i] = jnp.sum(hist[pl.ds(i * L, L)])

            def s2(i, carry):
                t = sums[i]
                sums[i] = carry
                return carry + t
            lax.fori_loop(0, NBUCKET, s2, np.int32(0))

            @plsc.parallel_loop(0, NBUCKET, unroll=4)
            def s3(i):
                v = hist[pl.ds(i * L, L)]
                hist[pl.ds(i * L, L)] = plsc.cumsum(v) - v + sums[i]

            for base, nch in ((0, NST0), (NST0, NST1)):
                if p == 0 or base > 0:
                    @plsc.parallel_loop(0, nch, unroll=4)
                    def stage(g):
                        abuf[pl.ds(g * L, L)] = addr_at(base + g)

                def bstep(g):
                    a = abuf[pl.ds(g * L, L)]
                    pos = plsc.load_gather(hist, [a])
                    plsc.addupdate_scatter(hist, [a], ones)
                    if p < NPASS - 1:
                        pos = ((pos & np.int32(SEG - 1)) << 4) | lax.shift_right_logical(pos, np.int32(SEG_SHIFT))
                    abuf[pl.ds(g * L, L)] = pos

                def bloop(g4, _):
                    for u in range(4):
                        bstep(g4 * 4 + u)
                    return _
                lax.fori_loop(0, nch // 4, bloop, 0)

                @plsc.parallel_loop(0, nch, unroll=4)
                def cloop(g):
                    ph = abuf[pl.ds(g * L, L)]
                    plsc.store_scatter(dst, [ph], iv_at(base + g))

        idx_dma = pltpu.async_copy(pa, idx_hbm.at[row], sem)

        @plsc.parallel_loop(0, NCHUNK, unroll=4)
        def ochunk(c):
            iv = pa[pl.ds(c * L, L)]
            kb = plsc.load_gather(key_v, [iv])
            pb[pl.ds(c * L, L)] = jnp.where(kb < 0, kb ^ INT_MIN, ~kb)

        idx_dma.wait()
        pltpu.sync_copy(pb, val_hbm.at[row])
        return row_carry

    lax.fori_loop(0, ROWS_PER_W, do_row, 0)


@jax.jit
def kernel(x):
    mesh = plsc.VectorSubcoreMesh(
        core_axis_name="c", subcore_axis_name="s", num_cores=NC, num_subcores=NS
    )
    run = pl.kernel(
        _body,
        out_type=(
            jax.ShapeDtypeStruct((R, N), jnp.int32),
            jax.ShapeDtypeStruct((R, N), jnp.int32),
        ),
        mesh=mesh,
        compiler_params=pltpu.CompilerParams(needs_layout_passes=False),
        scratch_types=[
            pltpu.VMEM((N,), jnp.int32),
            pltpu.VMEM((N,), jnp.int32),
            pltpu.VMEM((N,), jnp.int32),
            pltpu.VMEM((HIST,), jnp.int32),
            pltpu.VMEM((NST0 * L,), jnp.int32),
            pltpu.SMEM((NBUCKET,), jnp.int32),
            pltpu.SemaphoreType.DMA,
        ],
    )
    val_bits, idx = run(lax.bitcast_convert_type(x, jnp.int32))
    return lax.bitcast_convert_type(val_bits, jnp.float32), idx


# speedup vs baseline: 1.0316x; 1.0316x over previous
import numpy as np

import jax
import jax.numpy as jnp
from jax import lax
from jax.experimental import pallas as pl
from jax.experimental.pallas import tpu as pltpu
from jax.experimental.pallas import tpu_sc as plsc

R = 128
N = 32768
L = 16
NC = 2
NS = 16
NW = NC * NS
ROWS_PER_W = R // NW
SEG = N // L
NCHUNK = N // L
NST0 = 3 * NCHUNK // 4
NST1 = NCHUNK - NST0
SEG_SHIFT = 11
NBITS = 8
NBUCKET = 1 << NBITS
NPASS = 32 // NBITS
HIST = NBUCKET * L
INT_MIN = np.int32(-2147483648)


def _body(x_hbm, val_hbm, idx_hbm, key_v, pa, pb, hist, abuf, sums, sem):
    wid = lax.axis_index("s") * NC + lax.axis_index("c")
    lane = lax.iota(jnp.int32, L)
    ones = jnp.ones((L,), jnp.int32)
    dmask = np.int32(NBUCKET - 1)

    def do_row(r, row_carry):
        row = wid * ROWS_PER_W + r
        pltpu.sync_copy(x_hbm.at[row], key_v)

        @plsc.parallel_loop(0, NBUCKET, unroll=8)
        def zero0(i):
            hist[pl.ds(i * L, L)] = jnp.zeros((L,), jnp.int32)

        @plsc.parallel_loop(0, NCHUNK, unroll=8)
        def tx(c):
            b = key_v[pl.ds(c * L, L)]
            b = jnp.where(b == INT_MIN, np.int32(0), b)
            k = jnp.where(b < 0, ~b, b ^ INT_MIN)
            key_v[pl.ds(c * L, L)] = k
            addr = (k & dmask) * L + lax.shift_right_logical(c, 7)
            plsc.addupdate_scatter(hist, [addr], ones)

        for p in range(NPASS):
            shift = np.int32(NBITS * p)
            src = (None, pb, pa, pb)[p]
            dst = (pb, pa, pb, pa)[p]

            def iv_at(c):
                if p == 0:
                    return lane * SEG + c
                return src[pl.ds(c * L, L)]

            def addr_at(c):
                kb = plsc.load_gather(key_v, [iv_at(c)])
                d = lax.shift_right_logical(kb, shift) & dmask
                return d * L + lane

            if p > 0:
                @plsc.parallel_loop(0, NBUCKET, unroll=8)
                def zero(i):
                    hist[pl.ds(i * L, L)] = jnp.zeros((L,), jnp.int32)

                @plsc.parallel_loop(0, NST0, unroll=8)
                def aloop0(c):
                    a = addr_at(c)
                    plsc.addupdate_scatter(hist, [a], ones)
                    abuf[pl.ds(c * L, L)] = a

                @plsc.parallel_loop(NST0, NCHUNK, unroll=8)
                def aloop1(c):
                    plsc.addupdate_scatter(hist, [addr_at(c)], ones)

            @plsc.parallel_loop(0, NBUCKET, unroll=8)
            def s1(i):
                sums[i] = jnp.sum(hist[pl.ds(i * L, L)])

            def s2(i, carry):
                t = sums[i]
                sums[i] = carry
                return carry + t
            lax.fori_loop(0, NBUCKET, s2, np.int32(0))

            @plsc.parallel_loop(0, NBUCKET, unroll=8)
            def s3(i):
                v = hist[pl.ds(i * L, L)]
                hist[pl.ds(i * L, L)] = plsc.cumsum(v) - v + sums[i]

            for base, nch in ((0, NST0), (NST0, NST1)):
                if p == 0 or base > 0:
                    @plsc.parallel_loop(0, nch, unroll=8)
                    def stage(g):
                        abuf[pl.ds(g * L, L)] = addr_at(base + g)

                def bstep(g):
                    a = abuf[pl.ds(g * L, L)]
                    pos = plsc.load_gather(hist, [a])
                    plsc.addupdate_scatter(hist, [a], ones)
                    if p < NPASS - 1:
                        pos = ((pos & np.int32(SEG - 1)) << 4) | lax.shift_right_logical(pos, np.int32(SEG_SHIFT))
                    abuf[pl.ds(g * L, L)] = pos

                def bloop(g4, _):
                    for u in range(8):
                        bstep(g4 * 8 + u)
                    return _
                lax.fori_loop(0, nch // 8, bloop, 0)

                @plsc.parallel_loop(0, nch, unroll=8)
                def cloop(g):
                    ph = abuf[pl.ds(g * L, L)]
                    plsc.store_scatter(dst, [ph], iv_at(base + g))

        idx_dma = pltpu.async_copy(pa, idx_hbm.at[row], sem)

        @plsc.parallel_loop(0, NCHUNK, unroll=8)
        def ochunk(c):
            iv = pa[pl.ds(c * L, L)]
            kb = plsc.load_gather(key_v, [iv])
            pb[pl.ds(c * L, L)] = jnp.where(kb < 0, kb ^ INT_MIN, ~kb)

        idx_dma.wait()
        pltpu.sync_copy(pb, val_hbm.at[row])
        return row_carry

    lax.fori_loop(0, ROWS_PER_W, do_row, 0)


@jax.jit
def kernel(x):
    mesh = plsc.VectorSubcoreMesh(
        core_axis_name="c", subcore_axis_name="s", num_cores=NC, num_subcores=NS
    )
    run = pl.kernel(
        _body,
        out_type=(
            jax.ShapeDtypeStruct((R, N), jnp.int32),
            jax.ShapeDtypeStruct((R, N), jnp.int32),
        ),
        mesh=mesh,
        compiler_params=pltpu.CompilerParams(needs_layout_passes=False),
        scratch_types=[
            pltpu.VMEM((N,), jnp.int32),
            pltpu.VMEM((N,), jnp.int32),
            pltpu.VMEM((N,), jnp.int32),
            pltpu.VMEM((HIST,), jnp.int32),
            pltpu.VMEM((NST0 * L,), jnp.int32),
            pltpu.SMEM((NBUCKET,), jnp.int32),
            pltpu.SemaphoreType.DMA,
        ],
    )
    val_bits, idx = run(lax.bitcast_convert_type(x, jnp.int32))
    return lax.bitcast_convert_type(val_bits, jnp.float32), idx
